# baseline (speedup 1.0000x reference)
import functools

import jax
import jax.numpy as jnp
from jax import lax
from jax.experimental import pallas as pl
from jax.experimental.pallas import tpu as pltpu
from jax.experimental.pallas import tpu_sc as plsc

N = 10000
E = 320000
D = 64

NC, NS = 2, 16
NW = NC * NS
EW = E // NW
BLK = 200
NBLK = EW // BLK
SUB = 40
NSUB = BLK // SUB
RPT = N // NS

_MESH = plsc.VectorSubcoreMesh(
    core_axis_name="c", subcore_axis_name="s", num_cores=NC, num_subcores=NS)
_SC_PARAMS = pltpu.CompilerParams(use_tc_tiling_on_sc=False,
                                  needs_layout_passes=False)

_f32 = jnp.float32


def _proj_body(x_ref, ws_ref, wd_ref, b_ref, ps_ref, pd_ref):
    x = x_ref[...]
    ps_ref[...] = jnp.dot(x, ws_ref[...], preferred_element_type=_f32)
    pd_ref[...] = (jnp.dot(x, wd_ref[...], preferred_element_type=_f32)
                   + b_ref[...])


def _tc_proj(nf, ws, wd, b):
    n, din = nf.shape
    bn_ = 1000
    return pl.pallas_call(
        _proj_body,
        grid=(n // bn_,),
        in_specs=[pl.BlockSpec((bn_, din), lambda i: (i, 0)),
                  pl.BlockSpec((din, D), lambda i: (0, 0)),
                  pl.BlockSpec((din, D), lambda i: (0, 0)),
                  pl.BlockSpec((1, D), lambda i: (0, 0))],
        out_specs=[pl.BlockSpec((bn_, D), lambda i: (i, 0)),
                   pl.BlockSpec((bn_, D), lambda i: (i, 0))],
        out_shape=[jax.ShapeDtypeStruct((n, D), _f32),
                   jax.ShapeDtypeStruct((n, D), _f32)],
        compiler_params=pltpu.CompilerParams(
            dimension_semantics=("parallel",)),
    )(nf, ws, wd, b)


def _edge_body(gs_ref, gd_ref, x_ref, w_ref, a_ref, e1_ref, lg_ref, m_ref):
    t = (gs_ref[...] + gd_ref[...]
         + jnp.dot(x_ref[...], w_ref[...], preferred_element_type=_f32))
    e1 = jnp.maximum(t, 0.0)
    e1_ref[...] = e1
    lg = jnp.dot(e1, a_ref[...], preferred_element_type=_f32)
    lg_ref[...] = lg
    prev = jnp.where(pl.program_id(0) == 0, -jnp.inf, m_ref[0, 0])
    m_ref[0, 0] = jnp.maximum(prev, jnp.max(lg))


def _tc_edge(gs, gd, x, wx, a):
    be_ = 2000
    din = x.shape[1]
    return pl.pallas_call(
        _edge_body,
        grid=(E // be_,),
        in_specs=[pl.BlockSpec((be_, D), lambda i: (i, 0)),
                  pl.BlockSpec((be_, D), lambda i: (i, 0)),
                  pl.BlockSpec((be_, din), lambda i: (i, 0)),
                  pl.BlockSpec((din, D), lambda i: (0, 0)),
                  pl.BlockSpec((D, 1), lambda i: (0, 0))],
        out_specs=[pl.BlockSpec((be_, D), lambda i: (i, 0)),
                   pl.BlockSpec((be_, 1), lambda i: (i, 0)),
                   pl.BlockSpec(memory_space=pltpu.SMEM)],
        out_shape=[jax.ShapeDtypeStruct((E, D), _f32),
                   jax.ShapeDtypeStruct((E, 1), _f32),
                   jax.ShapeDtypeStruct((1, 1), _f32)],
    )(gs, gd, x, wx, a)


def _scale_body(e1_ref, lg_ref, m_ref, sw_ref):
    be_ = e1_ref.shape[0]
    num = jnp.exp(lg_ref[...] - m_ref[0, 0])
    sw_ref[...] = jnp.concatenate(
        [e1_ref[...] * num, jnp.broadcast_to(num, (be_, D))], axis=1)


def _tc_scale(e1, lg, m):
    be_ = 2000
    return pl.pallas_call(
        _scale_body,
        grid=(E // be_,),
        in_specs=[pl.BlockSpec((be_, D), lambda i: (i, 0)),
                  pl.BlockSpec((be_, 1), lambda i: (i, 0)),
                  pl.BlockSpec(memory_space=pltpu.SMEM)],
        out_specs=pl.BlockSpec((be_, 2 * D), lambda i: (i, 0)),
        out_shape=jax.ShapeDtypeStruct((E, 2 * D), _f32),
        compiler_params=pltpu.CompilerParams(
            dimension_semantics=("parallel",)),
    )(e1, lg, m)


def _node_new(nf_ref, ag_ref, wt_ref, wb_ref, b_ref):
    agg = ag_ref[0] + ag_ref[1]
    den = agg[:, D:D + 1]
    aggv = agg[:, :D] / (den + 1e-9)
    t = (jnp.dot(nf_ref[...], wt_ref[...], preferred_element_type=_f32)
         + jnp.dot(aggv, wb_ref[...], preferred_element_type=_f32)
         + b_ref[...])
    return jnp.maximum(t, 0.0)


def _node_body(nf_ref, ag_ref, wt_ref, wb_ref, b_ref, o_ref):
    o_ref[...] = _node_new(nf_ref, ag_ref, wt_ref, wb_ref, b_ref)


def _node_proj_body(nf_ref, ag_ref, wt_ref, wb_ref, b_ref,
                    ws_ref, wd_ref, b2_ref, o_ref, ps_ref, pd_ref):
    o = _node_new(nf_ref, ag_ref, wt_ref, wb_ref, b_ref)
    o_ref[...] = o
    ps_ref[...] = jnp.dot(o, ws_ref[...], preferred_element_type=_f32)
    pd_ref[...] = (jnp.dot(o, wd_ref[...], preferred_element_type=_f32)
                   + b2_ref[...])


def _node_in_specs(din, bn_):
    return [pl.BlockSpec((bn_, din), lambda i: (i, 0)),
            pl.BlockSpec((NC, bn_, 2 * D), lambda i: (0, i, 0)),
            pl.BlockSpec((din, D), lambda i: (0, 0)),
            pl.BlockSpec((D, D), lambda i: (0, 0)),
            pl.BlockSpec((1, D), lambda i: (0, 0))]


def _tc_node(nf, aggp, wt, wb, b):
    din = nf.shape[1]
    bn_ = 1000
    return pl.pallas_call(
        _node_body,
        grid=(N // bn_,),
        in_specs=_node_in_specs(din, bn_),
        out_specs=pl.BlockSpec((bn_, D), lambda i: (i, 0)),
        out_shape=jax.ShapeDtypeStruct((N, D), _f32),
        compiler_params=pltpu.CompilerParams(
            dimension_semantics=("parallel",)),
    )(nf, aggp, wt, wb, b)


def _tc_node_proj(nf, aggp, wt, wb, b, ws2, wd2, b2):
    din = nf.shape[1]
    bn_ = 1000
    return pl.pallas_call(
        _node_proj_body,
        grid=(N // bn_,),
        in_specs=_node_in_specs(din, bn_) + [
            pl.BlockSpec((D, D), lambda i: (0, 0)),
            pl.BlockSpec((D, D), lambda i: (0, 0)),
            pl.BlockSpec((1, D), lambda i: (0, 0))],
        out_specs=[pl.BlockSpec((bn_, D), lambda i: (i, 0)),
                   pl.BlockSpec((bn_, D), lambda i: (i, 0)),
                   pl.BlockSpec((bn_, D), lambda i: (i, 0))],
        out_shape=[jax.ShapeDtypeStruct((N, D), _f32),
                   jax.ShapeDtypeStruct((N, D), _f32),
                   jax.ShapeDtypeStruct((N, D), _f32)],
        compiler_params=pltpu.CompilerParams(
            dimension_semantics=("parallel",)),
    )(nf, aggp, wt, wb, b, ws2, wd2, b2)


def _sc_gather_body(ps_hbm, pd_hbm, src_hbm, dst_hbm, gs_hbm, gd_hbm,
                    sidx, didx, rs, rd, sem0, sem1):
    c = lax.axis_index("c")
    s = lax.axis_index("s")
    wbase = (c * NS + s) * EW

    def block(k, carry):
        gbase = wbase + k * BLK
        for j in range(NSUB):
            pltpu.sync_copy(src_hbm.at[pl.ds(gbase + j * SUB, SUB)],
                            sidx.at[j])
            pltpu.sync_copy(dst_hbm.at[pl.ds(gbase + j * SUB, SUB)],
                            didx.at[j])
        cps = [pltpu.async_copy(ps_hbm.at[sidx.at[j]],
                                rs.at[pl.ds(j * SUB, SUB)], sem0)
               for j in range(NSUB)]
        cpd = [pltpu.async_copy(pd_hbm.at[didx.at[j]],
                                rd.at[pl.ds(j * SUB, SUB)], sem1)
               for j in range(NSUB)]
        for cp in cps + cpd:
            cp.wait()
        pltpu.sync_copy(rs, gs_hbm.at[pl.ds(gbase, BLK)])
        pltpu.sync_copy(rd, gd_hbm.at[pl.ds(gbase, BLK)])
        return carry

    lax.fori_loop(0, NBLK, block, 0)


def _sc_gather(ps, pd, src, dst):
    return pl.kernel(
        _sc_gather_body,
        out_type=[jax.ShapeDtypeStruct((E, D), _f32),
                  jax.ShapeDtypeStruct((E, D), _f32)],
        mesh=_MESH,
        scratch_types=[pltpu.VMEM((NSUB, SUB), jnp.int32),
                       pltpu.VMEM((NSUB, SUB), jnp.int32),
                       pltpu.VMEM((BLK, D), _f32),
                       pltpu.VMEM((BLK, D), _f32),
                       pltpu.SemaphoreType.DMA,
                       pltpu.SemaphoreType.DMA],
        compiler_params=_SC_PARAMS,
    )(ps, pd, src, dst)


CH = 125
NCH = RPT // CH
SBLK = 200
NSBLK = EW // SBLK
SSUB = 40
NSSUB = SBLK // SSUB


def _sc_scatter_body(sw_hbm, dst_hbm, agg_hbm, didx, eb, zb, agg_sh):
    c = lax.axis_index("c")
    s = lax.axis_index("s")
    wbase = (c * NS + s) * EW
    zv = jnp.zeros((16,), _f32)

    def zrow(r, carry):
        for k in range(2 * D // 16):
            zb[r, pl.ds(k * 16, 16)] = zv
        return carry

    lax.fori_loop(0, CH, zrow, 0)
    for h in range(NCH):
        pltpu.sync_copy(zb, agg_sh.at[pl.ds(s * RPT + h * CH, CH)])
    plsc.subcore_barrier()

    def block(k, carry):
        gbase = wbase + k * SBLK
        for j in range(NSSUB):
            pltpu.sync_copy(dst_hbm.at[pl.ds(gbase + j * SSUB, SSUB)],
                            didx.at[j])
        pltpu.sync_copy(sw_hbm.at[pl.ds(gbase, SBLK)], eb)
        for j in range(NSSUB):
            pltpu.sync_copy(eb.at[pl.ds(j * SSUB, SSUB)],
                            agg_sh.at[didx.at[j]], add=True)
        return carry

    lax.fori_loop(0, NSBLK, block, 0)
    plsc.subcore_barrier()

    for h in range(NCH):
        pltpu.sync_copy(agg_sh.at[pl.ds(s * RPT + h * CH, CH)], zb)
        pltpu.sync_copy(zb, agg_hbm.at[c, pl.ds(s * RPT + h * CH, CH)])


def _sc_scatter(sw, dst):
    return pl.kernel(
        _sc_scatter_body,
        out_type=jax.ShapeDtypeStruct((NC, N, 2 * D), _f32),
        mesh=_MESH,
        scratch_types=[pltpu.VMEM((NSSUB, SSUB), jnp.int32),
                       pltpu.VMEM((SBLK, 2 * D), _f32),
                       pltpu.VMEM((CH, 2 * D), _f32),
                       pltpu.VMEM_SHARED((N, 2 * D), _f32)],
        compiler_params=_SC_PARAMS,
    )(sw, dst)


def kernel(nf, ef, We1, be1, a1, Wn1, bn1, We2, be2, a2, Wn2, bn2, edge_index):
    src = edge_index[0].astype(jnp.int32)
    dst = edge_index[1].astype(jnp.int32)
    dn1 = nf.shape[1]

    ps1, pd1 = _tc_proj(nf, We1[:dn1], We1[dn1:2 * dn1], be1.reshape(1, D))
    gs1, gd1 = _sc_gather(ps1, pd1, src, dst)
    e1, lg1, m1 = _tc_edge(gs1, gd1, ef, We1[2 * dn1:], a1.reshape(D, 1))
    sw1 = _tc_scale(e1, lg1, m1)
    aggp1 = _sc_scatter(sw1, dst)
    nf1, ps2, pd2 = _tc_node_proj(
        nf, aggp1, Wn1[:dn1], Wn1[dn1:], bn1.reshape(1, D),
        We2[:D], We2[D:2 * D], be2.reshape(1, D))

    gs2, gd2 = _sc_gather(ps2, pd2, src, dst)
    e2, lg2, m2 = _tc_edge(gs2, gd2, e1, We2[2 * D:], a2.reshape(D, 1))
    sw2 = _tc_scale(e2, lg2, m2)
    aggp2 = _sc_scatter(sw2, dst)
    nf2 = _tc_node(nf1, aggp2, Wn2[:D], Wn2[D:], bn2.reshape(1, D))
    return (nf2, e2)


# speedup vs baseline: 1.0807x; 1.0807x over previous
import functools

import jax
import jax.numpy as jnp
from jax import lax
from jax.experimental import pallas as pl
from jax.experimental.pallas import tpu as pltpu
from jax.experimental.pallas import tpu_sc as plsc

N = 10000
E = 320000
D = 64

NC, NS = 2, 16
NW = NC * NS
EW = E // NW
BLK = 400
NBLK = EW // BLK
SUB = 80
NSUB = BLK // SUB
RPT = N // NS

_MESH = plsc.VectorSubcoreMesh(
    core_axis_name="c", subcore_axis_name="s", num_cores=NC, num_subcores=NS)
_SC_PARAMS = pltpu.CompilerParams(use_tc_tiling_on_sc=False,
                                  needs_layout_passes=False)

_f32 = jnp.float32


def _proj_body(x_ref, ws_ref, wd_ref, b_ref, ps_ref, pd_ref):
    x = x_ref[...]
    ps_ref[...] = jnp.dot(x, ws_ref[...], preferred_element_type=_f32)
    pd_ref[...] = (jnp.dot(x, wd_ref[...], preferred_element_type=_f32)
                   + b_ref[...])


def _tc_proj(nf, ws, wd, b):
    n, din = nf.shape
    bn_ = 1000
    return pl.pallas_call(
        _proj_body,
        grid=(n // bn_,),
        in_specs=[pl.BlockSpec((bn_, din), lambda i: (i, 0)),
                  pl.BlockSpec((din, D), lambda i: (0, 0)),
                  pl.BlockSpec((din, D), lambda i: (0, 0)),
                  pl.BlockSpec((1, D), lambda i: (0, 0))],
        out_specs=[pl.BlockSpec((bn_, D), lambda i: (i, 0)),
                   pl.BlockSpec((bn_, D), lambda i: (i, 0))],
        out_shape=[jax.ShapeDtypeStruct((n, D), _f32),
                   jax.ShapeDtypeStruct((n, D), _f32)],
        compiler_params=pltpu.CompilerParams(
            dimension_semantics=("parallel",)),
    )(nf, ws, wd, b)


def _edge_body(gs_ref, gd_ref, x_ref, w_ref, a_ref, e1_ref, lg_ref, m_ref):
    t = (gs_ref[...] + gd_ref[...]
         + jnp.dot(x_ref[...], w_ref[...], preferred_element_type=_f32))
    e1 = jnp.maximum(t, 0.0)
    e1_ref[...] = e1
    lg = jnp.dot(e1, a_ref[...], preferred_element_type=_f32)
    lg_ref[...] = lg
    prev = jnp.where(pl.program_id(0) == 0, -jnp.inf, m_ref[0, 0])
    m_ref[0, 0] = jnp.maximum(prev, jnp.max(lg))


def _tc_edge(gs, gd, x, wx, a):
    be_ = 2000
    din = x.shape[1]
    return pl.pallas_call(
        _edge_body,
        grid=(E // be_,),
        in_specs=[pl.BlockSpec((be_, D), lambda i: (i, 0)),
                  pl.BlockSpec((be_, D), lambda i: (i, 0)),
                  pl.BlockSpec((be_, din), lambda i: (i, 0)),
                  pl.BlockSpec((din, D), lambda i: (0, 0)),
                  pl.BlockSpec((D, 1), lambda i: (0, 0))],
        out_specs=[pl.BlockSpec((be_, D), lambda i: (i, 0)),
                   pl.BlockSpec((be_, 1), lambda i: (i, 0)),
                   pl.BlockSpec(memory_space=pltpu.SMEM)],
        out_shape=[jax.ShapeDtypeStruct((E, D), _f32),
                   jax.ShapeDtypeStruct((E, 1), _f32),
                   jax.ShapeDtypeStruct((1, 1), _f32)],
    )(gs, gd, x, wx, a)


def _scale_body(e1_ref, lg_ref, m_ref, sw_ref):
    be_ = e1_ref.shape[0]
    num = jnp.exp(lg_ref[...] - m_ref[0, 0])
    sw_ref[...] = jnp.concatenate(
        [e1_ref[...] * num, jnp.broadcast_to(num, (be_, D))], axis=1)


def _tc_scale(e1, lg, m):
    be_ = 2000
    return pl.pallas_call(
        _scale_body,
        grid=(E // be_,),
        in_specs=[pl.BlockSpec((be_, D), lambda i: (i, 0)),
                  pl.BlockSpec((be_, 1), lambda i: (i, 0)),
                  pl.BlockSpec(memory_space=pltpu.SMEM)],
        out_specs=pl.BlockSpec((be_, 2 * D), lambda i: (i, 0)),
        out_shape=jax.ShapeDtypeStruct((E, 2 * D), _f32),
        compiler_params=pltpu.CompilerParams(
            dimension_semantics=("parallel",)),
    )(e1, lg, m)


def _node_new(nf_ref, ag_ref, wt_ref, wb_ref, b_ref):
    agg = ag_ref[0] + ag_ref[1]
    den = agg[:, D:D + 1]
    aggv = agg[:, :D] / (den + 1e-9)
    t = (jnp.dot(nf_ref[...], wt_ref[...], preferred_element_type=_f32)
         + jnp.dot(aggv, wb_ref[...], preferred_element_type=_f32)
         + b_ref[...])
    return jnp.maximum(t, 0.0)


def _node_body(nf_ref, ag_ref, wt_ref, wb_ref, b_ref, o_ref):
    o_ref[...] = _node_new(nf_ref, ag_ref, wt_ref, wb_ref, b_ref)


def _node_proj_body(nf_ref, ag_ref, wt_ref, wb_ref, b_ref,
                    ws_ref, wd_ref, b2_ref, o_ref, ps_ref, pd_ref):
    o = _node_new(nf_ref, ag_ref, wt_ref, wb_ref, b_ref)
    o_ref[...] = o
    ps_ref[...] = jnp.dot(o, ws_ref[...], preferred_element_type=_f32)
    pd_ref[...] = (jnp.dot(o, wd_ref[...], preferred_element_type=_f32)
                   + b2_ref[...])


def _node_in_specs(din, bn_):
    return [pl.BlockSpec((bn_, din), lambda i: (i, 0)),
            pl.BlockSpec((NC, bn_, 2 * D), lambda i: (0, i, 0)),
            pl.BlockSpec((din, D), lambda i: (0, 0)),
            pl.BlockSpec((D, D), lambda i: (0, 0)),
            pl.BlockSpec((1, D), lambda i: (0, 0))]


def _tc_node(nf, aggp, wt, wb, b):
    din = nf.shape[1]
    bn_ = 1000
    return pl.pallas_call(
        _node_body,
        grid=(N // bn_,),
        in_specs=_node_in_specs(din, bn_),
        out_specs=pl.BlockSpec((bn_, D), lambda i: (i, 0)),
        out_shape=jax.ShapeDtypeStruct((N, D), _f32),
        compiler_params=pltpu.CompilerParams(
            dimension_semantics=("parallel",)),
    )(nf, aggp, wt, wb, b)


def _tc_node_proj(nf, aggp, wt, wb, b, ws2, wd2, b2):
    din = nf.shape[1]
    bn_ = 1000
    return pl.pallas_call(
        _node_proj_body,
        grid=(N // bn_,),
        in_specs=_node_in_specs(din, bn_) + [
            pl.BlockSpec((D, D), lambda i: (0, 0)),
            pl.BlockSpec((D, D), lambda i: (0, 0)),
            pl.BlockSpec((1, D), lambda i: (0, 0))],
        out_specs=[pl.BlockSpec((bn_, D), lambda i: (i, 0)),
                   pl.BlockSpec((bn_, D), lambda i: (i, 0)),
                   pl.BlockSpec((bn_, D), lambda i: (i, 0))],
        out_shape=[jax.ShapeDtypeStruct((N, D), _f32),
                   jax.ShapeDtypeStruct((N, D), _f32),
                   jax.ShapeDtypeStruct((N, D), _f32)],
        compiler_params=pltpu.CompilerParams(
            dimension_semantics=("parallel",)),
    )(nf, aggp, wt, wb, b, ws2, wd2, b2)


def _sc_gather_body(ps_hbm, pd_hbm, src_hbm, dst_hbm, gs_hbm, gd_hbm,
                    sidx, didx, rs, rd, sem0, sem1):
    c = lax.axis_index("c")
    s = lax.axis_index("s")
    wbase = (c * NS + s) * EW

    def block(k, carry):
        gbase = wbase + k * BLK
        for j in range(NSUB):
            pltpu.sync_copy(src_hbm.at[pl.ds(gbase + j * SUB, SUB)],
                            sidx.at[j])
            pltpu.sync_copy(dst_hbm.at[pl.ds(gbase + j * SUB, SUB)],
                            didx.at[j])
        cps = [pltpu.async_copy(ps_hbm.at[sidx.at[j]],
                                rs.at[pl.ds(j * SUB, SUB)], sem0)
               for j in range(NSUB)]
        cpd = [pltpu.async_copy(pd_hbm.at[didx.at[j]],
                                rd.at[pl.ds(j * SUB, SUB)], sem1)
               for j in range(NSUB)]
        for cp in cps + cpd:
            cp.wait()
        pltpu.sync_copy(rs, gs_hbm.at[pl.ds(gbase, BLK)])
        pltpu.sync_copy(rd, gd_hbm.at[pl.ds(gbase, BLK)])
        return carry

    lax.fori_loop(0, NBLK, block, 0)


def _sc_gather(ps, pd, src, dst):
    return pl.kernel(
        _sc_gather_body,
        out_type=[jax.ShapeDtypeStruct((E, D), _f32),
                  jax.ShapeDtypeStruct((E, D), _f32)],
        mesh=_MESH,
        scratch_types=[pltpu.VMEM((NSUB, SUB), jnp.int32),
                       pltpu.VMEM((NSUB, SUB), jnp.int32),
                       pltpu.VMEM((BLK, D), _f32),
                       pltpu.VMEM((BLK, D), _f32),
                       pltpu.SemaphoreType.DMA,
                       pltpu.SemaphoreType.DMA],
        compiler_params=_SC_PARAMS,
    )(ps, pd, src, dst)


CH = 125
NCH = RPT // CH
SBLK = 200
NSBLK = EW // SBLK
SSUB = 40
NSSUB = SBLK // SSUB


def _sc_scatter_body(sw_hbm, dst_hbm, agg_hbm, didx, eb, zb, agg_sh):
    c = lax.axis_index("c")
    s = lax.axis_index("s")
    wbase = (c * NS + s) * EW
    zv = jnp.zeros((16,), _f32)

    def zrow(r, carry):
        for k in range(2 * D // 16):
            zb[r, pl.ds(k * 16, 16)] = zv
        return carry

    lax.fori_loop(0, CH, zrow, 0)
    for h in range(NCH):
        pltpu.sync_copy(zb, agg_sh.at[pl.ds(s * RPT + h * CH, CH)])
    plsc.subcore_barrier()

    def block(k, carry):
        gbase = wbase + k * SBLK
        for j in range(NSSUB):
            pltpu.sync_copy(dst_hbm.at[pl.ds(gbase + j * SSUB, SSUB)],
                            didx.at[j])
        pltpu.sync_copy(sw_hbm.at[pl.ds(gbase, SBLK)], eb)
        for j in range(NSSUB):
            pltpu.sync_copy(eb.at[pl.ds(j * SSUB, SSUB)],
                            agg_sh.at[didx.at[j]], add=True)
        return carry

    lax.fori_loop(0, NSBLK, block, 0)
    plsc.subcore_barrier()

    for h in range(NCH):
        pltpu.sync_copy(agg_sh.at[pl.ds(s * RPT + h * CH, CH)], zb)
        pltpu.sync_copy(zb, agg_hbm.at[c, pl.ds(s * RPT + h * CH, CH)])


def _sc_scatter(sw, dst):
    return pl.kernel(
        _sc_scatter_body,
        out_type=jax.ShapeDtypeStruct((NC, N, 2 * D), _f32),
        mesh=_MESH,
        scratch_types=[pltpu.VMEM((NSSUB, SSUB), jnp.int32),
                       pltpu.VMEM((SBLK, 2 * D), _f32),
                       pltpu.VMEM((CH, 2 * D), _f32),
                       pltpu.VMEM_SHARED((N, 2 * D), _f32)],
        compiler_params=_SC_PARAMS,
    )(sw, dst)


def kernel(nf, ef, We1, be1, a1, Wn1, bn1, We2, be2, a2, Wn2, bn2, edge_index):
    src = edge_index[0].astype(jnp.int32)
    dst = edge_index[1].astype(jnp.int32)
    dn1 = nf.shape[1]

    ps1, pd1 = _tc_proj(nf, We1[:dn1], We1[dn1:2 * dn1], be1.reshape(1, D))
    gs1, gd1 = _sc_gather(ps1, pd1, src, dst)
    e1, lg1, m1 = _tc_edge(gs1, gd1, ef, We1[2 * dn1:], a1.reshape(D, 1))
    sw1 = _tc_scale(e1, lg1, m1)
    aggp1 = _sc_scatter(sw1, dst)
    nf1, ps2, pd2 = _tc_node_proj(
        nf, aggp1, Wn1[:dn1], Wn1[dn1:], bn1.reshape(1, D),
        We2[:D], We2[D:2 * D], be2.reshape(1, D))

    gs2, gd2 = _sc_gather(ps2, pd2, src, dst)
    e2, lg2, m2 = _tc_edge(gs2, gd2, e1, We2[2 * D:], a2.reshape(D, 1))
    sw2 = _tc_scale(e2, lg2, m2)
    aggp2 = _sc_scatter(sw2, dst)
    nf2 = _tc_node(nf1, aggp2, Wn2[:D], Wn2[D:], bn2.reshape(1, D))
    return (nf2, e2)


# speedup vs baseline: 1.7045x; 1.5773x over previous
import functools

import jax
import jax.numpy as jnp
from jax import lax
from jax.experimental import pallas as pl
from jax.experimental.pallas import tpu as pltpu
from jax.experimental.pallas import tpu_sc as plsc

N = 10000
E = 320000
D = 64

NC, NS = 2, 16
NW = NC * NS
EW = E // NW
BLK = 400
NBLK = EW // BLK
SUB = 80
NSUB = BLK // SUB
RPT = N // NS

_MESH = plsc.VectorSubcoreMesh(
    core_axis_name="c", subcore_axis_name="s", num_cores=NC, num_subcores=NS)
_SC_PARAMS = pltpu.CompilerParams(use_tc_tiling_on_sc=False,
                                  needs_layout_passes=False)

_f32 = jnp.float32


def _proj_body(x_ref, ws_ref, wd_ref, b_ref, ps_ref, pd_ref):
    x = x_ref[...]
    ps_ref[...] = jnp.dot(x, ws_ref[...], preferred_element_type=_f32)
    pd_ref[...] = (jnp.dot(x, wd_ref[...], preferred_element_type=_f32)
                   + b_ref[...])


def _tc_proj(nf, ws, wd, b):
    n, din = nf.shape
    bn_ = 1000
    return pl.pallas_call(
        _proj_body,
        grid=(n // bn_,),
        in_specs=[pl.BlockSpec((bn_, din), lambda i: (i, 0)),
                  pl.BlockSpec((din, D), lambda i: (0, 0)),
                  pl.BlockSpec((din, D), lambda i: (0, 0)),
                  pl.BlockSpec((1, D), lambda i: (0, 0))],
        out_specs=[pl.BlockSpec((bn_, D), lambda i: (i, 0)),
                   pl.BlockSpec((bn_, D), lambda i: (i, 0))],
        out_shape=[jax.ShapeDtypeStruct((n, D), _f32),
                   jax.ShapeDtypeStruct((n, D), _f32)],
        compiler_params=pltpu.CompilerParams(
            dimension_semantics=("parallel",)),
    )(nf, ws, wd, b)


def _edge_body(g_ref, x_ref, w_ref, a_ref, e1_ref, lg_ref, m_ref):
    g = g_ref[...]
    t = (g[:, :D] + g[:, D:]
         + jnp.dot(x_ref[...], w_ref[...], preferred_element_type=_f32))
    e1 = jnp.maximum(t, 0.0)
    e1_ref[...] = e1
    lg = jnp.dot(e1, a_ref[...], preferred_element_type=_f32)
    lg_ref[...] = lg
    prev = jnp.where(pl.program_id(0) == 0, -jnp.inf, m_ref[0, 0])
    m_ref[0, 0] = jnp.maximum(prev, jnp.max(lg))


def _tc_edge(g, x, wx, a):
    be_ = 2000
    din = x.shape[1]
    return pl.pallas_call(
        _edge_body,
        grid=(E // be_,),
        in_specs=[pl.BlockSpec((be_, 2 * D), lambda i: (i, 0)),
                  pl.BlockSpec((be_, din), lambda i: (i, 0)),
                  pl.BlockSpec((din, D), lambda i: (0, 0)),
                  pl.BlockSpec((D, 1), lambda i: (0, 0))],
        out_specs=[pl.BlockSpec((be_, D), lambda i: (i, 0)),
                   pl.BlockSpec((be_, 1), lambda i: (i, 0)),
                   pl.BlockSpec(memory_space=pltpu.SMEM)],
        out_shape=[jax.ShapeDtypeStruct((E, D), _f32),
                   jax.ShapeDtypeStruct((E, 1), _f32),
                   jax.ShapeDtypeStruct((1, 1), _f32)],
    )(g, x, wx, a)


def _scale_body(e1_ref, lg_ref, m_ref, sw_ref):
    be_ = e1_ref.shape[0]
    num = jnp.exp(lg_ref[...] - m_ref[0, 0])
    sw_ref[...] = jnp.concatenate(
        [e1_ref[...] * num, jnp.broadcast_to(num, (be_, D))], axis=1)


def _tc_scale(e1, lg, m):
    be_ = 2000
    return pl.pallas_call(
        _scale_body,
        grid=(E // be_,),
        in_specs=[pl.BlockSpec((be_, D), lambda i: (i, 0)),
                  pl.BlockSpec((be_, 1), lambda i: (i, 0)),
                  pl.BlockSpec(memory_space=pltpu.SMEM)],
        out_specs=pl.BlockSpec((be_, 2 * D), lambda i: (i, 0)),
        out_shape=jax.ShapeDtypeStruct((E, 2 * D), _f32),
        compiler_params=pltpu.CompilerParams(
            dimension_semantics=("parallel",)),
    )(e1, lg, m)


def _node_new(nf_ref, ag_ref, wt_ref, wb_ref, b_ref):
    agg = ag_ref[0] + ag_ref[1]
    den = agg[:, D:D + 1]
    aggv = agg[:, :D] / (den + 1e-9)
    t = (jnp.dot(nf_ref[...], wt_ref[...], preferred_element_type=_f32)
         + jnp.dot(aggv, wb_ref[...], preferred_element_type=_f32)
         + b_ref[...])
    return jnp.maximum(t, 0.0)


def _node_body(nf_ref, ag_ref, wt_ref, wb_ref, b_ref, o_ref):
    o_ref[...] = _node_new(nf_ref, ag_ref, wt_ref, wb_ref, b_ref)


def _node_proj_body(nf_ref, ag_ref, wt_ref, wb_ref, b_ref,
                    ws_ref, wd_ref, b2_ref, o_ref, ps_ref, pd_ref):
    o = _node_new(nf_ref, ag_ref, wt_ref, wb_ref, b_ref)
    o_ref[...] = o
    ps_ref[...] = jnp.dot(o, ws_ref[...], preferred_element_type=_f32)
    pd_ref[...] = (jnp.dot(o, wd_ref[...], preferred_element_type=_f32)
                   + b2_ref[...])


def _node_in_specs(din, bn_):
    return [pl.BlockSpec((bn_, din), lambda i: (i, 0)),
            pl.BlockSpec((NC, bn_, 2 * D), lambda i: (0, i, 0)),
            pl.BlockSpec((din, D), lambda i: (0, 0)),
            pl.BlockSpec((D, D), lambda i: (0, 0)),
            pl.BlockSpec((1, D), lambda i: (0, 0))]


def _tc_node(nf, aggp, wt, wb, b):
    din = nf.shape[1]
    bn_ = 1000
    return pl.pallas_call(
        _node_body,
        grid=(N // bn_,),
        in_specs=_node_in_specs(din, bn_),
        out_specs=pl.BlockSpec((bn_, D), lambda i: (i, 0)),
        out_shape=jax.ShapeDtypeStruct((N, D), _f32),
        compiler_params=pltpu.CompilerParams(
            dimension_semantics=("parallel",)),
    )(nf, aggp, wt, wb, b)


def _tc_node_proj(nf, aggp, wt, wb, b, ws2, wd2, b2):
    din = nf.shape[1]
    bn_ = 1000
    return pl.pallas_call(
        _node_proj_body,
        grid=(N // bn_,),
        in_specs=_node_in_specs(din, bn_) + [
            pl.BlockSpec((D, D), lambda i: (0, 0)),
            pl.BlockSpec((D, D), lambda i: (0, 0)),
            pl.BlockSpec((1, D), lambda i: (0, 0))],
        out_specs=[pl.BlockSpec((bn_, D), lambda i: (i, 0)),
                   pl.BlockSpec((bn_, D), lambda i: (i, 0)),
                   pl.BlockSpec((bn_, D), lambda i: (i, 0))],
        out_shape=[jax.ShapeDtypeStruct((N, D), _f32),
                   jax.ShapeDtypeStruct((N, D), _f32),
                   jax.ShapeDtypeStruct((N, D), _f32)],
        compiler_params=pltpu.CompilerParams(
            dimension_semantics=("parallel",)),
    )(nf, aggp, wt, wb, b, ws2, wd2, b2)


def _sc_gather_body(ps_hbm, pd_hbm, src_hbm, dst_hbm, g_hbm,
                    sidx, didx, rs, rd, sem0, sem1):
    c = lax.axis_index("c")
    s = lax.axis_index("s")
    wbase = (c * NS + s) * EW
    pltpu.sync_copy(src_hbm.at[pl.ds(wbase, EW)], sidx)
    pltpu.sync_copy(dst_hbm.at[pl.ds(wbase, EW)], didx)

    def block(k, carry):
        gbase = wbase + k * BLK
        lbase = k * BLK
        cps = [pltpu.async_copy(
                   ps_hbm.at[sidx.at[pl.ds(lbase + j * SUB, SUB)]],
                   rs.at[pl.ds(j * SUB, SUB)], sem0)
               for j in range(NSUB)]
        cpd = [pltpu.async_copy(
                   pd_hbm.at[didx.at[pl.ds(lbase + j * SUB, SUB)]],
                   rd.at[pl.ds(j * SUB, SUB)], sem1)
               for j in range(NSUB)]
        for cp in cps + cpd:
            cp.wait()
        pltpu.sync_copy(rs, g_hbm.at[pl.ds(gbase, BLK), pl.ds(0, D)])
        pltpu.sync_copy(rd, g_hbm.at[pl.ds(gbase, BLK), pl.ds(D, D)])
        return carry

    lax.fori_loop(0, NBLK, block, 0)


def _sc_gather(ps, pd, src, dst):
    return pl.kernel(
        _sc_gather_body,
        out_type=jax.ShapeDtypeStruct((E, 2 * D), _f32),
        mesh=_MESH,
        scratch_types=[pltpu.VMEM((EW,), jnp.int32),
                       pltpu.VMEM((EW,), jnp.int32),
                       pltpu.VMEM((BLK, D), _f32),
                       pltpu.VMEM((BLK, D), _f32),
                       pltpu.SemaphoreType.DMA,
                       pltpu.SemaphoreType.DMA],
        compiler_params=_SC_PARAMS,
    )(ps, pd, src, dst)


CH = 25
NCH = RPT // CH
SBLK = 200
NSBLK = EW // SBLK
SSUB = 40
NSSUB = SBLK // SSUB


def _sc_scatter_body(sw_hbm, dst_hbm, agg_hbm, didx, eb, zb, agg_sh):
    c = lax.axis_index("c")
    s = lax.axis_index("s")
    wbase = (c * NS + s) * EW
    zv = jnp.zeros((16,), _f32)

    def zrow(r, carry):
        for k in range(2 * D // 16):
            zb[r, pl.ds(k * 16, 16)] = zv
        return carry

    lax.fori_loop(0, CH, zrow, 0)
    for h in range(NCH):
        pltpu.sync_copy(zb, agg_sh.at[pl.ds(s * RPT + h * CH, CH)])
    plsc.subcore_barrier()

    pltpu.sync_copy(dst_hbm.at[pl.ds(wbase, EW)], didx)

    def block(k, carry):
        gbase = wbase + k * SBLK
        lbase = k * SBLK
        pltpu.sync_copy(sw_hbm.at[pl.ds(gbase, SBLK)], eb)
        for j in range(NSSUB):
            pltpu.sync_copy(
                eb.at[pl.ds(j * SSUB, SSUB)],
                agg_sh.at[didx.at[pl.ds(lbase + j * SSUB, SSUB)]],
                add=True)
        return carry

    lax.fori_loop(0, NSBLK, block, 0)
    plsc.subcore_barrier()

    for h in range(NCH):
        pltpu.sync_copy(agg_sh.at[pl.ds(s * RPT + h * CH, CH)], zb)
        pltpu.sync_copy(zb, agg_hbm.at[c, pl.ds(s * RPT + h * CH, CH)])


def _sc_scatter(sw, dst):
    return pl.kernel(
        _sc_scatter_body,
        out_type=jax.ShapeDtypeStruct((NC, N, 2 * D), _f32),
        mesh=_MESH,
        scratch_types=[pltpu.VMEM((EW,), jnp.int32),
                       pltpu.VMEM((SBLK, 2 * D), _f32),
                       pltpu.VMEM((CH, 2 * D), _f32),
                       pltpu.VMEM_SHARED((N, 2 * D), _f32)],
        compiler_params=_SC_PARAMS,
    )(sw, dst)


def kernel(nf, ef, We1, be1, a1, Wn1, bn1, We2, be2, a2, Wn2, bn2, edge_index):
    src = edge_index[0].astype(jnp.int32)
    dst = edge_index[1].astype(jnp.int32)
    dn1 = nf.shape[1]

    ps1, pd1 = _tc_proj(nf, We1[:dn1], We1[dn1:2 * dn1], be1.reshape(1, D))
    g1 = _sc_gather(ps1, pd1, src, dst)
    e1, lg1, m1 = _tc_edge(g1, ef, We1[2 * dn1:], a1.reshape(D, 1))
    sw1 = _tc_scale(e1, lg1, m1)
    aggp1 = _sc_scatter(sw1, dst)
    nf1, ps2, pd2 = _tc_node_proj(
        nf, aggp1, Wn1[:dn1], Wn1[dn1:], bn1.reshape(1, D),
        We2[:D], We2[D:2 * D], be2.reshape(1, D))

    g2 = _sc_gather(ps2, pd2, src, dst)
    e2, lg2, m2 = _tc_edge(g2, e1, We2[2 * D:], a2.reshape(D, 1))
    sw2 = _tc_scale(e2, lg2, m2)
    aggp2 = _sc_scatter(sw2, dst)
    nf2 = _tc_node(nf1, aggp2, Wn2[:D], Wn2[D:], bn2.reshape(1, D))
    return (nf2, e2)
